# baseline (speedup 1.0000x reference)
import functools

import jax
import jax.numpy as jnp
from jax import lax
from jax.experimental import pallas as pl
from jax.experimental.pallas import tpu as pltpu, tpu_sc as plsc

TOKENS = 32768
DIM = 4096
EXPERTS = 64
K = 8
BLOCK_T = 256

NW = 32
ROWS_W = TOKENS // NW
CH = 128
NCH = ROWS_W // CH


def _matmul_body(x_ref, wt_ref, b_ref, p_ref):
    s = jnp.dot(x_ref[...].astype(jnp.bfloat16),
                wt_ref[...].astype(jnp.bfloat16),
                preferred_element_type=jnp.float32)
    s = s + b_ref[...]
    m = jnp.max(s, axis=1, keepdims=True)
    e = jnp.exp(s - m)
    p_ref[...] = e / jnp.sum(e, axis=1, keepdims=True)


def _sc_topk_body(p_hbm, idx_hbm, val_hbm, in_v, oi_v, ov_v):
    wid = lax.axis_index("s") * 2 + lax.axis_index("c")
    base = wid * ROWS_W
    lane = lax.iota(jnp.int32, 16)
    low8 = lane < 8

    def chunk_body(c, carry):
        row0 = base + c * CH
        pltpu.sync_copy(p_hbm.at[pl.ds(row0 * EXPERTS, CH * EXPERTS)], in_v)

        def row_body(r, carry2):
            def srt(g, desc):
                kk = in_v[pl.ds(r * EXPERTS + g * 16, 16)]
                return plsc.sort_key_val(kk, lane + g * 16, descending=desc)

            k0, v0 = srt(0, True)
            k1, v1 = srt(1, False)
            k2, v2 = srt(2, True)
            k3, v3 = srt(3, False)
            mk01 = jnp.where(low8, k0, k1)
            mv01 = jnp.where(low8, v0, v1)
            mk23 = jnp.where(low8, k2, k3)
            mv23 = jnp.where(low8, v2, v3)
            tk01, tv01 = plsc.sort_key_val(mk01, mv01, descending=True)
            tk23, tv23 = plsc.sort_key_val(mk23, mv23, descending=False)
            fk = jnp.where(low8, tk01, tk23)
            fv = jnp.where(low8, tv01, tv23)
            sk, sv = plsc.sort_key_val(fk, fv, descending=True)
            plsc.store_compressed(ov_v.at[pl.ds(r * K, 16)], sk, mask=low8)
            plsc.store_compressed(oi_v.at[pl.ds(r * K, 16)], sv, mask=low8)
            return carry2

        lax.fori_loop(0, CH, row_body, 0)
        pltpu.sync_copy(oi_v.at[pl.ds(0, CH * K)],
                        idx_hbm.at[pl.ds(row0 * K, CH * K)])
        pltpu.sync_copy(ov_v.at[pl.ds(0, CH * K)],
                        val_hbm.at[pl.ds(row0 * K, CH * K)])
        return carry

    lax.fori_loop(0, NCH, chunk_body, 0)


_sc_topk = functools.partial(
    pl.kernel,
    out_type=[
        jax.ShapeDtypeStruct((TOKENS * K,), jnp.int32),
        jax.ShapeDtypeStruct((TOKENS * K,), jnp.float32),
    ],
    mesh=plsc.VectorSubcoreMesh(core_axis_name="c", subcore_axis_name="s"),
    compiler_params=pltpu.CompilerParams(needs_layout_passes=False),
    scratch_types=[
        pltpu.VMEM((CH * EXPERTS,), jnp.float32),
        pltpu.VMEM((CH * K + 8,), jnp.int32),
        pltpu.VMEM((CH * K + 8,), jnp.float32),
    ],
)(_sc_topk_body)


@jax.jit
def kernel(x, W, b):
    wt = W.T
    b2 = b.reshape(1, EXPERTS)
    probs = pl.pallas_call(
        _matmul_body,
        grid=(TOKENS // BLOCK_T,),
        in_specs=[
            pl.BlockSpec((BLOCK_T, DIM), lambda i: (i, 0)),
            pl.BlockSpec((DIM, EXPERTS), lambda i: (0, 0)),
            pl.BlockSpec((1, EXPERTS), lambda i: (0, 0)),
        ],
        out_specs=pl.BlockSpec((BLOCK_T, EXPERTS), lambda i: (i, 0)),
        out_shape=jax.ShapeDtypeStruct((TOKENS, EXPERTS), jnp.float32),
    )(x, wt, b2)
    idx_flat, val_flat = _sc_topk(probs.reshape(-1))
    return idx_flat.reshape(TOKENS, K), val_flat.reshape(TOKENS, K)


# speedup vs baseline: 1.1023x; 1.1023x over previous
import functools

import jax
import jax.numpy as jnp
from jax import lax
from jax.experimental import pallas as pl
from jax.experimental.pallas import tpu as pltpu, tpu_sc as plsc

TOKENS = 32768
DIM = 4096
EXPERTS = 64
K = 8
BLOCK_T = 256

NCHUNK = 4
CT = TOKENS // NCHUNK

NW = 32
ROWS_W = CT // NW
CH = 128
NCH = ROWS_W // CH


def _matmul_body(x_ref, wt_ref, b_ref, p_ref):
    s = jnp.dot(x_ref[...].astype(jnp.bfloat16),
                wt_ref[...].astype(jnp.bfloat16),
                preferred_element_type=jnp.float32)
    s = s + b_ref[...]
    m = jnp.max(s, axis=1, keepdims=True)
    e = jnp.exp(s - m)
    p_ref[...] = e / jnp.sum(e, axis=1, keepdims=True)


def _sc_topk_body(p_hbm, idx_hbm, val_hbm, in_v, oi_v, ov_v):
    wid = lax.axis_index("s") * 2 + lax.axis_index("c")
    base = wid * ROWS_W
    lane = lax.iota(jnp.int32, 16)
    low8 = lane < 8

    def chunk_body(c, carry):
        row0 = base + c * CH
        pltpu.sync_copy(p_hbm.at[pl.ds(row0 * EXPERTS, CH * EXPERTS)], in_v)

        def row_body(r, carry2):
            def srt(g, desc):
                kk = in_v[pl.ds(r * EXPERTS + g * 16, 16)]
                return plsc.sort_key_val(kk, lane + g * 16, descending=desc)

            k0, v0 = srt(0, True)
            k1, v1 = srt(1, False)
            k2, v2 = srt(2, True)
            k3, v3 = srt(3, False)
            mk01 = jnp.where(low8, k0, k1)
            mv01 = jnp.where(low8, v0, v1)
            mk23 = jnp.where(low8, k2, k3)
            mv23 = jnp.where(low8, v2, v3)
            tk01, tv01 = plsc.sort_key_val(mk01, mv01, descending=True)
            tk23, tv23 = plsc.sort_key_val(mk23, mv23, descending=False)
            fk = jnp.where(low8, tk01, tk23)
            fv = jnp.where(low8, tv01, tv23)
            sk, sv = plsc.sort_key_val(fk, fv, descending=True)
            plsc.store_compressed(ov_v.at[pl.ds(r * K, 16)], sk, mask=low8)
            plsc.store_compressed(oi_v.at[pl.ds(r * K, 16)], sv, mask=low8)
            return carry2

        lax.fori_loop(0, CH, row_body, 0)
        pltpu.sync_copy(oi_v.at[pl.ds(0, CH * K)],
                        idx_hbm.at[pl.ds(row0 * K, CH * K)])
        pltpu.sync_copy(ov_v.at[pl.ds(0, CH * K)],
                        val_hbm.at[pl.ds(row0 * K, CH * K)])
        return carry

    lax.fori_loop(0, NCH, chunk_body, 0)


_sc_topk = functools.partial(
    pl.kernel,
    out_type=[
        jax.ShapeDtypeStruct((CT * K,), jnp.int32),
        jax.ShapeDtypeStruct((CT * K,), jnp.float32),
    ],
    mesh=plsc.VectorSubcoreMesh(core_axis_name="c", subcore_axis_name="s"),
    compiler_params=pltpu.CompilerParams(needs_layout_passes=False),
    scratch_types=[
        pltpu.VMEM((CH * EXPERTS,), jnp.float32),
        pltpu.VMEM((CH * K + 8,), jnp.int32),
        pltpu.VMEM((CH * K + 8,), jnp.float32),
    ],
)(_sc_topk_body)


@jax.jit
def kernel(x, W, b):
    wt = W.T
    b2 = b.reshape(1, EXPERTS)
    idx_parts = []
    val_parts = []
    for ci in range(NCHUNK):
        probs = pl.pallas_call(
            _matmul_body,
            grid=(CT // BLOCK_T,),
            in_specs=[
                pl.BlockSpec((BLOCK_T, DIM),
                             functools.partial(lambda c, i: (c + i, 0),
                                               ci * (CT // BLOCK_T))),
                pl.BlockSpec((DIM, EXPERTS), lambda i: (0, 0)),
                pl.BlockSpec((1, EXPERTS), lambda i: (0, 0)),
            ],
            out_specs=pl.BlockSpec((BLOCK_T, EXPERTS), lambda i: (i, 0)),
            out_shape=jax.ShapeDtypeStruct((CT, EXPERTS), jnp.float32),
        )(x, wt, b2)
        idx_flat, val_flat = _sc_topk(probs.reshape(-1))
        idx_parts.append(idx_flat.reshape(CT, K))
        val_parts.append(val_flat.reshape(CT, K))
    return (jnp.concatenate(idx_parts, axis=0),
            jnp.concatenate(val_parts, axis=0))
